# Initial kernel scaffold; baseline (speedup 1.0000x reference)
import jax
import jax.numpy as jnp
from jax.experimental import pallas as pl


def kernel(frame_input, W_score, b_score, W_combine, b_combine):
    raise NotImplementedError("write your pallas kernel here")


# speedup vs baseline: 1.7156x; 1.7156x over previous
import functools

import jax
import jax.numpy as jnp
from jax import lax
from jax.experimental import pallas as pl
import jax.experimental.pallas.tpu as pltpu
from jax.experimental.pallas import tpu_sc as plsc

B, S, D = 8, 4096, 512
BS = 512
NS = S // BS
CHUNK = 1024
NTILES = 32
TPB = 4
GROUPS = CHUNK // 16
CH = 112
RB = CH // 16
NDUMMY = 8
L16 = 16


def _stage1_body(thr_ref, x_ref, ws_ref, wc_ref, z_ref, start_ref,
                 carry_z, carry_a):
    ns = pl.program_id(1)
    x = x_ref[0]
    a2 = jnp.dot(x, ws_ref[:D, :], preferred_element_type=jnp.float32)
    c2 = jnp.dot(x, ws_ref[D:, :], preferred_element_type=jnp.float32)
    y = jnp.dot(x, wc_ref[...], preferred_element_type=jnp.float32)

    @pl.when(ns == 0)
    def _():
        carry_z[...] = jnp.zeros_like(carry_z)
        carry_a[...] = jnp.zeros_like(carry_a)

    z = y
    sh = 1
    while sh < BS:
        z = z + jnp.concatenate(
            [jnp.zeros((sh, D), jnp.float32), z[:-sh]], axis=0)
        sh *= 2
    z = z + carry_z[0:1, :]
    z_ref[0] = z
    carry_z[...] = z[BS - 1:BS, :]

    a_prev = jnp.concatenate([carry_a[0:1, :], a2[:-1, :]], axis=0)
    pair_prev = a_prev + c2
    startv = (pair_prev <= thr_ref[0]).astype(jnp.int32)
    row = lax.broadcasted_iota(jnp.int32, (BS, 1), 0)
    startv = jnp.where(jnp.logical_and(ns == 0, row == 0), 1, startv)
    start_ref[0] = startv
    carry_a[...] = a2[BS - 1:BS, :]


def _stage1(frame_input, W_score, b_score, W_combine):
    thr = (0.5 - b_score).astype(jnp.float32)
    return pl.pallas_call(
        _stage1_body,
        grid=(B, NS),
        in_specs=[
            pl.BlockSpec(memory_space=pltpu.SMEM),
            pl.BlockSpec((1, BS, D), lambda b, ns: (b, ns, 0)),
            pl.BlockSpec((2 * D, 1), lambda b, ns: (0, 0)),
            pl.BlockSpec((D, D), lambda b, ns: (0, 0)),
        ],
        out_specs=[
            pl.BlockSpec((1, BS, D), lambda b, ns: (b, ns, 0)),
            pl.BlockSpec((1, BS, 1), lambda b, ns: (b, ns, 0)),
        ],
        out_shape=[
            jax.ShapeDtypeStruct((B, S, D), jnp.float32),
            jax.ShapeDtypeStruct((B, S, 1), jnp.int32),
        ],
        scratch_shapes=[
            pltpu.VMEM((1, D), jnp.float32),
            pltpu.VMEM((1, 1), jnp.float32),
        ],
        compiler_params=pltpu.CompilerParams(
            dimension_semantics=("arbitrary", "arbitrary")),
    )(thr, frame_input, W_score, W_combine)


def _iota16():
    return lax.iota(jnp.int32, L16)


def _lane(vec, i):
    return jnp.sum(jnp.where(_iota16() == i, vec, 0))


def _sc_body(z_hbm, start_hbm, bias_hbm, out_hbm,
             start_v, e_v, rows_v, outb_v, dst_v, recip_v, rec1_v, rec_v,
             bias_v, rec_s, sem):
    cid = lax.axis_index("c")
    sid = lax.axis_index("s")
    b = cid * TPB + sid // TPB
    tb = sid % TPB
    f0 = b * S + tb * CHUNK
    pb0 = tb * CHUNK
    iota = _iota16()

    pltpu.sync_copy(bias_hbm, bias_v)
    pltpu.sync_copy(start_hbm.at[pl.ds(f0, CHUNK + 16)], start_v)

    for t in range((CHUNK + 16) // 16):
        e_v[pl.ds(16 * t, 16)] = iota + 16 * t

    def groupA(g, carry):
        off, last = carry
        sv = plsc.load_gather(start_v, [iota + (16 * g + 1)])
        bpos = iota + (pb0 + 16 * g)
        mask = jnp.logical_or(sv != 0, bpos == S - 1)
        posg = iota + (f0 + 16 * g)
        rank = plsc.cumsum(mask.astype(jnp.int32))
        plsc.store_scatter(e_v, [off + rank], posg, mask=mask)
        cnt = jnp.sum(mask.astype(jnp.int32))
        lastg = jnp.max(jnp.where(mask, posg, -1))
        return off + cnt, jnp.maximum(last, lastg)

    k, last = lax.fori_loop(0, GROUPS, groupA,
                            (jnp.int32(1), jnp.int32(-1)))
    k = k - 1

    rec1_v[...] = jnp.where(iota == 0, k, jnp.where(iota == 1, last, 0))
    pltpu.sync_copy(rec1_v, rec_s.at[sid])
    plsc.subcore_barrier()
    pltpu.sync_copy(rec_s, rec_v)

    jp = jnp.int32(0)
    prev = jnp.int32(-1)
    nseg = jnp.int32(0)
    my_bgrp = sid // TPB
    for t in range(16):
        rv = rec_v[t]
        kt = _lane(rv, 0)
        lt = _lane(rv, 1)
        in_b = (t // TPB) == my_bgrp
        prior = jnp.logical_and(in_b, t < sid)
        jp = jp + jnp.where(prior, kt, 0)
        prev = jnp.maximum(prev, jnp.where(prior, lt, -1))
        nseg = nseg + jnp.where(in_b, kt, 0)

    plsc.store_scatter(e_v, [iota * 0], jnp.maximum(prev, 0) + iota * 0,
                       mask=iota == 0)
    zero_first = prev < 0

    def chunkB(c, _):
        cpy = pltpu.make_async_copy(
            z_hbm.at[e_v.at[pl.ds(c * CH, CH + 1)]], rows_v, sem)
        cpy.start()
        cpy.wait()

        @pl.when(jnp.logical_and(c == 0, zero_first))
        def _():
            for d in range(D // 16):
                rows_v[0, pl.ds(16 * d, 16)] = jnp.zeros((16,), jnp.float32)

        for rb in range(RB):
            gv0 = plsc.load_gather(e_v, [iota + (c * CH + 16 * rb)])
            gv1 = plsc.load_gather(e_v, [iota + (c * CH + 16 * rb + 1)])
            recip_v[pl.ds(16 * rb, 16)] = 1.0 / (gv1 - gv0).astype(jnp.float32)
            lidx = iota + (c * CH + 16 * rb)
            dstv = jnp.where(lidx < k, b * S + jp + lidx,
                             B * S + ((sid * 2 + iota) & (NDUMMY - 1)))
            dst_v[0, pl.ds(16 * rb, 16)] = dstv

        def rowfn(r, _):
            recr = plsc.load_gather(recip_v, [iota * 0 + r])
            for d in range(D // 16):
                hi = rows_v[r + 1, pl.ds(16 * d, 16)]
                lo = rows_v[r, pl.ds(16 * d, 16)]
                outb_v[r, pl.ds(16 * d, 16)] = (
                    (hi - lo) * recr + bias_v[pl.ds(16 * d, 16)])
            return 0

        lax.fori_loop(0, CH, rowfn, 0)

        ocpy = pltpu.make_async_copy(outb_v, out_hbm.at[dst_v.at[0]], sem)
        ocpy.start()
        ocpy.wait()
        return 0

    nch = (k + CH - 1) // CH
    lax.fori_loop(0, nch, chunkB, 0)

    def biasrow(r, _):
        for d in range(D // 16):
            outb_v[r, pl.ds(16 * d, 16)] = bias_v[pl.ds(16 * d, 16)]
        return 0

    lax.fori_loop(0, CH, biasrow, 0)

    ltail = S - nseg
    m_t = jnp.maximum(0, (ltail - tb + TPB - 1) // TPB)

    def tailfn(mc, _):
        for rb in range(RB):
            midx = iota + (mc * CH + 16 * rb)
            dstv = jnp.where(midx < m_t,
                             b * S + nseg + tb + TPB * midx,
                             B * S + ((sid * 2 + iota) & (NDUMMY - 1)))
            dst_v[0, pl.ds(16 * rb, 16)] = dstv
        tcpy = pltpu.make_async_copy(outb_v, out_hbm.at[dst_v.at[0]], sem)
        tcpy.start()
        tcpy.wait()
        return 0

    nmch = (m_t + CH - 1) // CH
    lax.fori_loop(0, nmch, tailfn, 0)


def _stage2(z2d, start_flat, b_combine):
    mesh = plsc.VectorSubcoreMesh(core_axis_name="c", subcore_axis_name="s")
    return pl.kernel(
        _sc_body,
        out_type=jax.ShapeDtypeStruct((B * S + NDUMMY, D), jnp.float32),
        mesh=mesh,
        scratch_types=[
            pltpu.VMEM((CHUNK + 16,), jnp.int32),
            pltpu.VMEM((CHUNK + 16,), jnp.int32),
            pltpu.VMEM((CH + 1, D), jnp.float32),
            pltpu.VMEM((CH, D), jnp.float32),
            pltpu.VMEM((1, CH), jnp.int32),
            pltpu.VMEM((CH,), jnp.float32),
            pltpu.VMEM((16,), jnp.int32),
            pltpu.VMEM((16, 16), jnp.int32),
            pltpu.VMEM((D,), jnp.float32),
            pltpu.VMEM_SHARED((16, 16), jnp.int32),
            pltpu.SemaphoreType.DMA,
        ],
        compiler_params=pltpu.CompilerParams(needs_layout_passes=False),
    )(z2d, start_flat, b_combine)


@jax.jit
def kernel(frame_input, W_score, b_score, W_combine, b_combine):
    z, start = _stage1(frame_input, W_score, b_score, W_combine)
    z2d = z.reshape(B * S, D)
    start_flat = jnp.pad(start.reshape(B * S), (0, 64), constant_values=1)
    out_full = _stage2(z2d, start_flat, b_combine.astype(jnp.float32))
    return out_full[:B * S].reshape(B, S, D)
